# Initial kernel scaffold; baseline (speedup 1.0000x reference)
import jax
import jax.numpy as jnp
from jax.experimental import pallas as pl


def kernel(inputs, W):
    raise NotImplementedError("write your pallas kernel here")


# speedup vs baseline: 3.5687x; 3.5687x over previous
import functools

import jax
import jax.numpy as jnp
from jax.experimental import pallas as pl
from jax.experimental.pallas import tpu as pltpu

COMMIT = 0.25


def _vq_body(x_ref, w_ref, out_ref, loss_ref, acc_ref, *, nsteps, denom):
    step = pl.program_id(0) * pl.num_programs(1) + pl.program_id(1)

    x = x_ref[0]
    w = w_ref[0]

    s = jnp.dot(x, w, preferred_element_type=jnp.float32)
    a = jnp.sum(x * x, axis=1, keepdims=True)
    b = jnp.sum(w * w, axis=0, keepdims=True)
    dist = (a - 2.0 * s) + b

    mn = jnp.min(dist, axis=1, keepdims=True)
    kiota = jax.lax.broadcasted_iota(jnp.int32, dist.shape, 1)
    kdim = dist.shape[1]
    idx = jnp.min(jnp.where(dist == mn, kiota, kdim), axis=1)

    onehot = (kiota == idx[:, None]).astype(jnp.float32)
    q = jax.lax.dot_general(
        onehot, w,
        dimension_numbers=(((1,), (1,)), ((), ())),
        precision=jax.lax.Precision.HIGHEST,
        preferred_element_type=jnp.float32)
    out_ref[0] = q

    part = jnp.sum((q - x) ** 2)

    @pl.when(step == 0)
    def _():
        acc_ref[0, 0] = 0.0

    acc_ref[0, 0] += part

    @pl.when(step == nsteps - 1)
    def _():
        loss_ref[0, 0] = acc_ref[0, 0] * ((1.0 + COMMIT) / denom)


@functools.partial(jax.jit, static_argnames=("tb",))
def _vq_tc(inputs, W, tb=512):
    F, T, D = inputs.shape
    K = W.shape[2]
    nt = T // tb
    grid = (F, nt)
    out, loss = pl.pallas_call(
        functools.partial(_vq_body, nsteps=F * nt, denom=float(F * T * D)),
        grid=grid,
        in_specs=[
            pl.BlockSpec((1, tb, D), lambda f, t: (f, t, 0)),
            pl.BlockSpec((1, D, K), lambda f, t: (f, 0, 0)),
        ],
        out_specs=[
            pl.BlockSpec((1, tb, D), lambda f, t: (f, t, 0)),
            pl.BlockSpec(memory_space=pltpu.SMEM),
        ],
        out_shape=[
            jax.ShapeDtypeStruct((F, T, D), jnp.float32),
            jax.ShapeDtypeStruct((1, 1), jnp.float32),
        ],
        scratch_shapes=[pltpu.SMEM((1, 1), jnp.float32)],
    )(inputs, W)
    return out, loss.reshape(())


def kernel(inputs, W):
    return _vq_tc(inputs, W)


# speedup vs baseline: 4.2397x; 1.1880x over previous
import functools

import jax
import jax.numpy as jnp
from jax import lax
from jax.experimental import pallas as pl
from jax.experimental.pallas import tpu as pltpu
from jax.experimental.pallas import tpu_sc as plsc

COMMIT = 0.25

_NC, _NS = 2, 16
_NW = _NC * _NS
_CH = 128


def _argmin_body(xm2_ref, x_ref, w_ref, idx_ref, loss_ref, acc_ref,
                 *, nsteps, denom, kcode):
    step = pl.program_id(0) * pl.num_programs(1) + pl.program_id(1)

    x = x_ref[0]
    w = w_ref[0]

    s2 = jnp.dot(xm2_ref[0], w, preferred_element_type=jnp.float32)
    a = jnp.sum(x * x, axis=1, keepdims=True)
    b = jnp.sum(w * w, axis=0, keepdims=True)
    dist = (a + s2) + b

    mn = jnp.min(dist, axis=1, keepdims=True)
    kiota = jax.lax.broadcasted_iota(jnp.int32, dist.shape, 1)
    kdim = dist.shape[1]
    idx = jnp.min(jnp.where(dist == mn, kiota, kdim), axis=1)
    idx_ref[0, 0, 0] = idx + pl.program_id(0) * kcode

    part = jnp.sum(mn)

    @pl.when(step == 0)
    def _():
        acc_ref[0, 0] = 0.0

    acc_ref[0, 0] += part

    @pl.when(step == nsteps - 1)
    def _():
        loss_ref[0, 0] = acc_ref[0, 0] * ((1.0 + COMMIT) / denom)


def _gather_body(table_hbm, idx_hbm, out_hbm, idx_v, buf0, buf1, gsem, ssem,
                 *, b_per_w, d):
    wid = lax.axis_index("s") * _NC + lax.axis_index("c")
    n_ch = b_per_w // _CH
    base = wid * b_per_w
    pltpu.sync_copy(idx_hbm.at[wid], idx_v)
    bufs = (buf0, buf1)
    gathers = [None] * n_ch
    scatters = [None] * n_ch
    gathers[0] = pltpu.async_copy(table_hbm.at[idx_v.at[0]], buf0, gsem)
    for c in range(n_ch):
        gathers[c].wait()
        scatters[c] = pltpu.make_async_copy(
            bufs[c % 2], out_hbm.at[pl.ds(base + c * _CH, _CH)], ssem)
        scatters[c].start()
        if c + 1 < n_ch:
            if c >= 1:
                scatters[c - 1].wait()
            gathers[c + 1] = pltpu.async_copy(
                table_hbm.at[idx_v.at[c + 1]], bufs[(c + 1) % 2], gsem)
    scatters[n_ch - 1].wait()


@functools.partial(jax.jit, static_argnames=("tb",))
def _vq(inputs, W, tb=512):
    F, T, D = inputs.shape
    K = W.shape[2]
    nt = T // tb
    B = F * T
    b_per_w = B // _NW

    idx, loss = pl.pallas_call(
        functools.partial(_argmin_body, nsteps=F * nt, denom=float(F * T * D),
                          kcode=K),
        grid=(F, nt),
        in_specs=[
            pl.BlockSpec((1, tb, D), lambda f, t: (f, t, 0)),
            pl.BlockSpec((1, tb, D), lambda f, t: (f, t, 0)),
            pl.BlockSpec((1, D, K), lambda f, t: (f, 0, 0)),
        ],
        out_specs=[
            pl.BlockSpec((1, 1, 1, tb), lambda f, t: (f, t, 0, 0)),
            pl.BlockSpec(memory_space=pltpu.SMEM),
        ],
        out_shape=[
            jax.ShapeDtypeStruct((F, nt, 1, tb), jnp.int32),
            jax.ShapeDtypeStruct((1, 1), jnp.float32),
        ],
        scratch_shapes=[pltpu.SMEM((1, 1), jnp.float32)],
    )(inputs * (-2.0), inputs, W)

    table = jnp.transpose(W, (0, 2, 1)).reshape(F * K, D)
    idx3 = idx.reshape(_NW, b_per_w // _CH, _CH)

    mesh = plsc.VectorSubcoreMesh(core_axis_name="c", subcore_axis_name="s")
    gather = pl.kernel(
        functools.partial(_gather_body, b_per_w=b_per_w, d=D),
        out_type=jax.ShapeDtypeStruct((B, D), jnp.float32),
        mesh=mesh,
        scratch_types=[
            pltpu.VMEM((b_per_w // _CH, _CH), jnp.int32),
            pltpu.VMEM((_CH, D), jnp.float32),
            pltpu.VMEM((_CH, D), jnp.float32),
            pltpu.SemaphoreType.DMA,
            pltpu.SemaphoreType.DMA,
        ],
    )
    out = gather(table, idx3)
    return out.reshape(F, T, D), loss.reshape(())


def kernel(inputs, W):
    return _vq(inputs, W)


# speedup vs baseline: 4.9347x; 1.1639x over previous
import functools

import jax
import jax.numpy as jnp
from jax import lax
from jax.experimental import pallas as pl
from jax.experimental.pallas import tpu as pltpu
from jax.experimental.pallas import tpu_sc as plsc

COMMIT = 0.25

_NC, _NS = 2, 16
_NW = _NC * _NS
_CH = 128


def _argmin_body(x_ref, w_ref, idx_ref, loss_ref, acc_ref, b_ref,
                 *, nsteps, denom, kcode):
    step = pl.program_id(0) * pl.num_programs(1) + pl.program_id(1)

    x = x_ref[0]
    w = w_ref[0]

    @pl.when(pl.program_id(1) == 0)
    def _():
        b_ref[...] = jnp.sum(w * w, axis=0, keepdims=True)

    s2 = jnp.dot(x * (-2.0), w, preferred_element_type=jnp.float32)
    a = jnp.sum(x * x, axis=1, keepdims=True)
    dist = (a + s2) + b_ref[...]

    mn = jnp.min(dist, axis=1, keepdims=True)
    kiota = jax.lax.broadcasted_iota(jnp.int32, dist.shape, 1)
    kdim = dist.shape[1]
    idx = jnp.min(jnp.where(dist == mn, kiota, kdim), axis=1)
    idx_ref[0, 0, 0] = idx + pl.program_id(0) * kcode

    part = jnp.sum(mn)

    @pl.when(step == 0)
    def _():
        acc_ref[0, 0] = 0.0

    acc_ref[0, 0] += part

    @pl.when(step == nsteps - 1)
    def _():
        loss_ref[0, 0] = acc_ref[0, 0] * ((1.0 + COMMIT) / denom)


def _gather_body(table_hbm, idx_hbm, out_hbm, idx_v, buf0, buf1, gsem, ssem,
                 *, b_per_w, d):
    wid = lax.axis_index("s") * _NC + lax.axis_index("c")
    n_ch = b_per_w // _CH
    base = wid * b_per_w
    pltpu.sync_copy(idx_hbm.at[wid], idx_v)
    bufs = (buf0, buf1)
    gathers = [None] * n_ch
    scatters = [None] * n_ch
    gathers[0] = pltpu.async_copy(table_hbm.at[idx_v.at[0]], buf0, gsem)
    for c in range(n_ch):
        gathers[c].wait()
        scatters[c] = pltpu.make_async_copy(
            bufs[c % 2], out_hbm.at[pl.ds(base + c * _CH, _CH)], ssem)
        scatters[c].start()
        if c + 1 < n_ch:
            if c >= 1:
                scatters[c - 1].wait()
            gathers[c + 1] = pltpu.async_copy(
                table_hbm.at[idx_v.at[c + 1]], bufs[(c + 1) % 2], gsem)
    scatters[n_ch - 1].wait()


@functools.partial(jax.jit, static_argnames=("tb",))
def _vq(inputs, W, tb=512):
    F, T, D = inputs.shape
    K = W.shape[2]
    nt = T // tb
    B = F * T
    b_per_w = B // _NW

    idx, loss = pl.pallas_call(
        functools.partial(_argmin_body, nsteps=F * nt, denom=float(F * T * D),
                          kcode=K),
        grid=(F, nt),
        in_specs=[
            pl.BlockSpec((1, tb, D), lambda f, t: (f, t, 0)),
            pl.BlockSpec((1, D, K), lambda f, t: (f, 0, 0)),
        ],
        out_specs=[
            pl.BlockSpec((1, 1, 1, tb), lambda f, t: (f, t, 0, 0)),
            pl.BlockSpec(memory_space=pltpu.SMEM),
        ],
        out_shape=[
            jax.ShapeDtypeStruct((F, nt, 1, tb), jnp.int32),
            jax.ShapeDtypeStruct((1, 1), jnp.float32),
        ],
        scratch_shapes=[pltpu.SMEM((1, 1), jnp.float32),
                        pltpu.VMEM((1, K), jnp.float32)],
    )(inputs, W)

    table = jnp.transpose(W, (0, 2, 1)).reshape(F * K, D)
    idx3 = idx.reshape(_NW, b_per_w // _CH, _CH)

    mesh = plsc.VectorSubcoreMesh(core_axis_name="c", subcore_axis_name="s")
    gather = pl.kernel(
        functools.partial(_gather_body, b_per_w=b_per_w, d=D),
        out_type=jax.ShapeDtypeStruct((B, D), jnp.float32),
        mesh=mesh,
        scratch_types=[
            pltpu.VMEM((b_per_w // _CH, _CH), jnp.int32),
            pltpu.VMEM((_CH, D), jnp.float32),
            pltpu.VMEM((_CH, D), jnp.float32),
            pltpu.SemaphoreType.DMA,
            pltpu.SemaphoreType.DMA,
        ],
    )
    out = gather(table, idx3)
    return out.reshape(F, T, D), loss.reshape(())


def kernel(inputs, W):
    return _vq(inputs, W)


# speedup vs baseline: 5.6278x; 1.1405x over previous
import functools

import jax
import jax.numpy as jnp
from jax import lax
from jax.experimental import pallas as pl
from jax.experimental.pallas import tpu as pltpu
from jax.experimental.pallas import tpu_sc as plsc

COMMIT = 0.25

_NC, _NS = 2, 16
_NW = _NC * _NS
_CH = 128


def _argmin_body(x_ref, w_ref, idx_ref, loss_ref, acc_ref, b_ref, kf_ref,
                 *, nsteps, denom, kcode):
    step = pl.program_id(0) * pl.num_programs(1) + pl.program_id(1)

    x = x_ref[0]
    w = w_ref[0]

    @pl.when(pl.program_id(1) == 0)
    def _():
        b_ref[...] = jnp.sum(w * w, axis=0, keepdims=True)

    s2 = jnp.dot(x * (-2.0), w, preferred_element_type=jnp.float32)
    a = jnp.sum(x * x, axis=1, keepdims=True)
    dist = (a + s2) + b_ref[...]

    @pl.when(step == 0)
    def _():
        kf_ref[...] = jax.lax.broadcasted_iota(
            jnp.int32, kf_ref.shape, 1).astype(jnp.float32)

    mn = jnp.min(dist, axis=1, keepdims=True)
    kdim = float(dist.shape[1])
    idxf = jnp.min(jnp.where(dist == mn, kf_ref[...], kdim), axis=1,
                   keepdims=True)
    idx_ref[0] = idxf.astype(jnp.int32) + pl.program_id(0) * kcode

    part = jnp.sum(mn)

    @pl.when(step == 0)
    def _():
        acc_ref[0, 0] = 0.0

    acc_ref[0, 0] += part

    @pl.when(step == nsteps - 1)
    def _():
        loss_ref[0, 0] = acc_ref[0, 0] * ((1.0 + COMMIT) / denom)


def _gather_body(table_hbm, idx_hbm, out_hbm, idx_v, buf0, buf1, gsem, ssem,
                 *, b_per_w, d):
    wid = lax.axis_index("s") * _NC + lax.axis_index("c")
    n_ch = b_per_w // _CH
    base = wid * b_per_w
    pltpu.sync_copy(idx_hbm.at[wid], idx_v)
    bufs = (buf0, buf1)
    gathers = [None] * n_ch
    scatters = [None] * n_ch
    gathers[0] = pltpu.async_copy(table_hbm.at[idx_v.at[0]], buf0, gsem)
    for c in range(n_ch):
        gathers[c].wait()
        scatters[c] = pltpu.make_async_copy(
            bufs[c % 2], out_hbm.at[pl.ds(base + c * _CH, _CH)], ssem)
        scatters[c].start()
        if c + 1 < n_ch:
            if c >= 1:
                scatters[c - 1].wait()
            gathers[c + 1] = pltpu.async_copy(
                table_hbm.at[idx_v.at[c + 1]], bufs[(c + 1) % 2], gsem)
    scatters[n_ch - 1].wait()


@functools.partial(jax.jit, static_argnames=("tb",))
def _vq(inputs, W, tb=512):
    F, T, D = inputs.shape
    K = W.shape[2]
    nt = T // tb
    B = F * T
    b_per_w = B // _NW

    idx, loss = pl.pallas_call(
        functools.partial(_argmin_body, nsteps=F * nt, denom=float(F * T * D),
                          kcode=K),
        grid=(F, nt),
        in_specs=[
            pl.BlockSpec((1, tb, D), lambda f, t: (f, t, 0)),
            pl.BlockSpec((1, D, K), lambda f, t: (f, 0, 0)),
        ],
        out_specs=[
            pl.BlockSpec((1, tb, 1), lambda f, t: (f, t, 0)),
            pl.BlockSpec(memory_space=pltpu.SMEM),
        ],
        out_shape=[
            jax.ShapeDtypeStruct((F, T, 1), jnp.int32),
            jax.ShapeDtypeStruct((1, 1), jnp.float32),
        ],
        scratch_shapes=[pltpu.SMEM((1, 1), jnp.float32),
                        pltpu.VMEM((1, K), jnp.float32),
                        pltpu.VMEM((tb, K), jnp.float32)],
    )(inputs, W)

    table = jnp.transpose(W, (0, 2, 1)).reshape(F * K, D)
    idx3 = idx.reshape(_NW, b_per_w // _CH, _CH)

    mesh = plsc.VectorSubcoreMesh(core_axis_name="c", subcore_axis_name="s")
    gather = pl.kernel(
        functools.partial(_gather_body, b_per_w=b_per_w, d=D),
        out_type=jax.ShapeDtypeStruct((B, D), jnp.float32),
        mesh=mesh,
        scratch_types=[
            pltpu.VMEM((b_per_w // _CH, _CH), jnp.int32),
            pltpu.VMEM((_CH, D), jnp.float32),
            pltpu.VMEM((_CH, D), jnp.float32),
            pltpu.SemaphoreType.DMA,
            pltpu.SemaphoreType.DMA,
        ],
    )
    out = gather(table, idx3)
    return out.reshape(F, T, D), loss.reshape(())


def kernel(inputs, W):
    return _vq(inputs, W)


# speedup vs baseline: 6.5691x; 1.1673x over previous
import functools

import jax
import jax.numpy as jnp
from jax import lax
from jax.experimental import pallas as pl
from jax.experimental.pallas import tpu as pltpu
from jax.experimental.pallas import tpu_sc as plsc

COMMIT = 0.25

_NC, _NS = 2, 16
_NW = _NC * _NS
_CH = 128


def _argmin_body(x_ref, w_ref, idx_ref, loss_ref, wt_ref, acc_ref, b_ref,
                 kf_ref, *, nsteps, denom, kcode):
    step = pl.program_id(0) * pl.num_programs(1) + pl.program_id(1)

    x = x_ref[0]
    w = w_ref[0]

    @pl.when(pl.program_id(1) == 0)
    def _():
        b_ref[...] = jnp.sum(w * w, axis=0, keepdims=True)
        wt_ref[0] = w.T

    s2 = jnp.dot(x * (-2.0), w, preferred_element_type=jnp.float32)
    a = jnp.sum(x * x, axis=1, keepdims=True)
    dist = (a + s2) + b_ref[...]

    @pl.when(step == 0)
    def _():
        kf_ref[...] = jax.lax.broadcasted_iota(
            jnp.int32, kf_ref.shape, 1).astype(jnp.float32)

    mn = jnp.min(dist, axis=1, keepdims=True)
    kdim = float(dist.shape[1])
    idxf = jnp.min(jnp.where(dist == mn, kf_ref[...], kdim), axis=1,
                   keepdims=True)
    idx_ref[0] = idxf.astype(jnp.int32) + pl.program_id(0) * kcode

    part = jnp.sum(mn)

    @pl.when(step == 0)
    def _():
        acc_ref[0, 0] = 0.0

    acc_ref[0, 0] += part

    @pl.when(step == nsteps - 1)
    def _():
        loss_ref[0, 0] = acc_ref[0, 0] * ((1.0 + COMMIT) / denom)


def _gather_body(table_hbm, idx_hbm, out_hbm, idx_v, buf0, buf1, gsem, ssem,
                 *, b_per_w, d):
    wid = lax.axis_index("s") * _NC + lax.axis_index("c")
    n_ch = b_per_w // _CH
    base = wid * b_per_w
    pltpu.sync_copy(idx_hbm.at[wid], idx_v)
    bufs = (buf0, buf1)
    gathers = [None] * n_ch
    scatters = [None] * n_ch
    gathers[0] = pltpu.async_copy(table_hbm.at[idx_v.at[0]], buf0, gsem)
    for c in range(n_ch):
        gathers[c].wait()
        scatters[c] = pltpu.make_async_copy(
            bufs[c % 2], out_hbm.at[pl.ds(base + c * _CH, _CH)], ssem)
        scatters[c].start()
        if c + 1 < n_ch:
            if c >= 1:
                scatters[c - 1].wait()
            gathers[c + 1] = pltpu.async_copy(
                table_hbm.at[idx_v.at[c + 1]], bufs[(c + 1) % 2], gsem)
    scatters[n_ch - 1].wait()


@functools.partial(jax.jit, static_argnames=("tb",))
def _vq(inputs, W, tb=1024):
    F, T, D = inputs.shape
    K = W.shape[2]
    nt = T // tb
    B = F * T
    b_per_w = B // _NW

    idx, loss, wt = pl.pallas_call(
        functools.partial(_argmin_body, nsteps=F * nt, denom=float(F * T * D),
                          kcode=K),
        grid=(F, nt),
        in_specs=[
            pl.BlockSpec((1, tb, D), lambda f, t: (f, t, 0)),
            pl.BlockSpec((1, D, K), lambda f, t: (f, 0, 0)),
        ],
        out_specs=[
            pl.BlockSpec((1, tb, 1), lambda f, t: (f, t, 0)),
            pl.BlockSpec(memory_space=pltpu.SMEM),
            pl.BlockSpec((1, K, D), lambda f, t: (f, 0, 0)),
        ],
        out_shape=[
            jax.ShapeDtypeStruct((F, T, 1), jnp.int32),
            jax.ShapeDtypeStruct((1, 1), jnp.float32),
            jax.ShapeDtypeStruct((F, K, D), jnp.float32),
        ],
        scratch_shapes=[pltpu.SMEM((1, 1), jnp.float32),
                        pltpu.VMEM((1, K), jnp.float32),
                        pltpu.VMEM((tb, K), jnp.float32)],
    )(inputs, W)

    table = wt.reshape(F * K, D)
    idx3 = idx.reshape(_NW, b_per_w // _CH, _CH)

    mesh = plsc.VectorSubcoreMesh(core_axis_name="c", subcore_axis_name="s")
    gather = pl.kernel(
        functools.partial(_gather_body, b_per_w=b_per_w, d=D),
        out_type=jax.ShapeDtypeStruct((B, D), jnp.float32),
        mesh=mesh,
        scratch_types=[
            pltpu.VMEM((b_per_w // _CH, _CH), jnp.int32),
            pltpu.VMEM((_CH, D), jnp.float32),
            pltpu.VMEM((_CH, D), jnp.float32),
            pltpu.SemaphoreType.DMA,
            pltpu.SemaphoreType.DMA,
        ],
    )
    out = gather(table, idx3)
    return out.reshape(F, T, D), loss.reshape(())


def kernel(inputs, W):
    return _vq(inputs, W)
